# baseline (speedup 1.0000x reference)
import jax
import jax.numpy as jnp
from jax import lax
from jax.experimental import pallas as pl
from jax.experimental.pallas import tpu as pltpu
from jax.experimental.pallas import tpu_sc as plsc

DIN = 512
DOUT = 1536
NG = 512
BATCH = 4096

NUM_WORKERS = 32
ROWS_PER_W = BATCH // NUM_WORKERS
RB = 16
NBLK = ROWS_PER_W // RB
GCH = NG // 16


def _fredkin_body(x_hbm, wgts_hbm, out_hbm, wgts_v, coef_v,
                  xb0, xb1, ob0, ob1, sin0, sin1, sout0, sout1):
    cid = lax.axis_index("c")
    sid = lax.axis_index("s")
    wid = sid * 2 + cid
    iota = lax.iota(jnp.int32, 16)

    pltpu.sync_copy(wgts_hbm, wgts_v)

    def build_coefs(j, _):
        g6 = (j * 16 + iota) * 6
        ws = [plsc.load_gather(wgts_v, [g6 + s]) for s in range(6)]
        mx = ws[0]
        for s in range(1, 6):
            mx = jnp.maximum(mx, ws[s])
        e = [jnp.exp(w - mx) for w in ws]
        inv = 1.0 / (e[0] + e[1] + e[2] + e[3] + e[4] + e[5])
        coefs = [
            (e[0] + e[1]) * inv, (e[2] + e[3]) * inv, (e[4] + e[5]) * inv,
            (e[3] + e[5]) * inv, (e[1] + e[4]) * inv, (e[0] + e[2]) * inv,
            (e[0] - e[1] + e[2] - e[3]) * inv,
            (-e[0] + e[1] + e[4] - e[5]) * inv,
            (-e[2] + e[3] - e[4] + e[5]) * inv,
        ]
        for k in range(9):
            coef_v[pl.ds(k * NG + j * 16, 16)] = coefs[k]
        return 0

    lax.fori_loop(0, GCH, build_coefs, 0)

    xbufs, obufs = [xb0, xb1], [ob0, ob1]
    sins, souts = [sin0, sin1], [sout0, sout1]

    def start_in(b):
        r0 = wid * ROWS_PER_W + b * RB
        return pltpu.async_copy(
            x_hbm.at[pl.ds(r0, RB), :], xbufs[b % 2], sins[b % 2])

    def start_out(b):
        r0 = wid * ROWS_PER_W + b * RB
        return pltpu.async_copy(
            obufs[b % 2], out_hbm.at[pl.ds(r0, RB), :], souts[b % 2])

    def compute_block(xb_v, ob_v):
        def do_chunk(j, _):
            g3 = (j * 16 + iota) * 3
            col0 = lax.rem(g3 + 1, DIN)
            col1 = lax.rem(g3 + 2, DIN)
            col2 = lax.rem(g3 + 3, DIN)
            cf = [coef_v[pl.ds(k * NG + j * 16, 16)] for k in range(9)]
            a0, a1, a2, b0, b1, b2, t3, t4, t5 = cf

            @plsc.parallel_loop(0, RB, unroll=8)
            def do_row(i):
                rs = jnp.full((16,), i, jnp.int32)
                x0 = plsc.load_gather(xb_v, [rs, col0])
                x1 = plsc.load_gather(xb_v, [rs, col1])
                x2 = plsc.load_gather(xb_v, [rs, col2])
                q = t3 * (x0 * x1) + t4 * (x0 * x2) + t5 * (x1 * x2)
                o0 = a0 * x0 + a1 * x1 + a2 * x2
                o1 = b0 * x0 + b1 * x1 + b2 * x2 + q
                o2 = (x0 + x1 + x2) - o0 - o1
                plsc.store_scatter(ob_v, [rs, g3], o0)
                plsc.store_scatter(ob_v, [rs, g3 + 1], o1)
                plsc.store_scatter(ob_v, [rs, g3 + 2], o2)

            return 0

        lax.fori_loop(0, GCH, do_chunk, 0)

    in_descs = [start_in(0)]
    out_descs = []
    for b in range(NBLK):
        if b + 1 < NBLK:
            in_descs.append(start_in(b + 1))
        in_descs[b].wait()
        if b >= 2:
            out_descs[b - 2].wait()
        compute_block(xbufs[b % 2], obufs[b % 2])
        out_descs.append(start_out(b))
    out_descs[NBLK - 2].wait()
    out_descs[NBLK - 1].wait()


def kernel(x, wgts):
    mesh = plsc.VectorSubcoreMesh(core_axis_name="c", subcore_axis_name="s")
    run = pl.kernel(
        _fredkin_body,
        out_type=jax.ShapeDtypeStruct((BATCH, DOUT), jnp.float32),
        mesh=mesh,
        compiler_params=pltpu.CompilerParams(needs_layout_passes=False),
        scratch_types=[
            pltpu.VMEM((NG * 6,), jnp.float32),
            pltpu.VMEM((9 * NG,), jnp.float32),
            pltpu.VMEM((RB, DIN), jnp.float32),
            pltpu.VMEM((RB, DIN), jnp.float32),
            pltpu.VMEM((RB, DOUT), jnp.float32),
            pltpu.VMEM((RB, DOUT), jnp.float32),
            pltpu.SemaphoreType.DMA,
            pltpu.SemaphoreType.DMA,
            pltpu.SemaphoreType.DMA,
            pltpu.SemaphoreType.DMA,
        ],
    )
    return run(x, wgts.astype(jnp.float32).reshape(-1))


# speedup vs baseline: 1.0179x; 1.0179x over previous
import jax
import jax.numpy as jnp
from jax import lax
from jax.experimental import pallas as pl
from jax.experimental.pallas import tpu as pltpu
from jax.experimental.pallas import tpu_sc as plsc

DIN = 512
DOUT = 1536
NG = 512
BATCH = 4096

NUM_WORKERS = 32
ROWS_PER_W = BATCH // NUM_WORKERS
RB = 16
NBLK = ROWS_PER_W // RB
GCH = NG // 16


def _fredkin_body(x_hbm, wgts_hbm, out_hbm, wgts_v, coef_v,
                  xb0, xb1, ob0, ob1, sin0, sin1, sout0, sout1):
    cid = lax.axis_index("c")
    sid = lax.axis_index("s")
    wid = sid * 2 + cid
    iota = lax.iota(jnp.int32, 16)

    pltpu.sync_copy(wgts_hbm, wgts_v)

    def build_coefs(j, _):
        g6 = (j * 16 + iota) * 6
        ws = [plsc.load_gather(wgts_v, [g6 + s]) for s in range(6)]
        mx = ws[0]
        for s in range(1, 6):
            mx = jnp.maximum(mx, ws[s])
        e = [jnp.exp(w - mx) for w in ws]
        inv = 1.0 / (e[0] + e[1] + e[2] + e[3] + e[4] + e[5])
        coefs = [
            (e[0] + e[1]) * inv, (e[2] + e[3]) * inv, (e[4] + e[5]) * inv,
            (e[3] + e[5]) * inv, (e[1] + e[4]) * inv, (e[0] + e[2]) * inv,
            (e[0] - e[1] + e[2] - e[3]) * inv,
            (-e[0] + e[1] + e[4] - e[5]) * inv,
            (-e[2] + e[3] - e[4] + e[5]) * inv,
        ]
        for k in range(9):
            coef_v[pl.ds(k * NG + j * 16, 16)] = coefs[k]
        return 0

    lax.fori_loop(0, GCH, build_coefs, 0)

    xbufs, obufs = [xb0, xb1], [ob0, ob1]
    sins, souts = [sin0, sin1], [sout0, sout1]

    def start_in(b):
        r0 = wid * ROWS_PER_W + b * RB
        return pltpu.async_copy(
            x_hbm.at[pl.ds(r0, RB), :], xbufs[b % 2], sins[b % 2])

    def start_out(b):
        r0 = wid * ROWS_PER_W + b * RB
        return pltpu.async_copy(
            obufs[b % 2], out_hbm.at[pl.ds(r0, RB), :], souts[b % 2])

    def compute_block(xb_v, ob_v):
        def do_chunk(j, _):
            g3 = (j * 16 + iota) * 3
            col0 = lax.rem(g3 + 1, DIN)
            col1 = lax.rem(g3 + 2, DIN)
            col2 = lax.rem(g3 + 3, DIN)
            cf = [coef_v[pl.ds(k * NG + j * 16, 16)] for k in range(9)]
            a0, a1, a2, b0, b1, b2, t3, t4, t5 = cf

            @plsc.parallel_loop(0, RB, unroll=2)
            def do_row(i):
                rs = jnp.full((16,), i, jnp.int32)
                x0 = plsc.load_gather(xb_v, [rs, col0])
                x1 = plsc.load_gather(xb_v, [rs, col1])
                x2 = plsc.load_gather(xb_v, [rs, col2])
                q = t3 * (x0 * x1) + t4 * (x0 * x2) + t5 * (x1 * x2)
                o0 = a0 * x0 + a1 * x1 + a2 * x2
                o1 = b0 * x0 + b1 * x1 + b2 * x2 + q
                o2 = (x0 + x1 + x2) - o0 - o1
                plsc.store_scatter(ob_v, [rs, g3], o0)
                plsc.store_scatter(ob_v, [rs, g3 + 1], o1)
                plsc.store_scatter(ob_v, [rs, g3 + 2], o2)

            return 0

        lax.fori_loop(0, GCH, do_chunk, 0)

    in_descs = [start_in(0)]
    out_descs = []
    for b in range(NBLK):
        if b + 1 < NBLK:
            in_descs.append(start_in(b + 1))
        in_descs[b].wait()
        if b >= 2:
            out_descs[b - 2].wait()
        compute_block(xbufs[b % 2], obufs[b % 2])
        out_descs.append(start_out(b))
    out_descs[NBLK - 2].wait()
    out_descs[NBLK - 1].wait()


def kernel(x, wgts):
    mesh = plsc.VectorSubcoreMesh(core_axis_name="c", subcore_axis_name="s")
    run = pl.kernel(
        _fredkin_body,
        out_type=jax.ShapeDtypeStruct((BATCH, DOUT), jnp.float32),
        mesh=mesh,
        compiler_params=pltpu.CompilerParams(needs_layout_passes=False),
        scratch_types=[
            pltpu.VMEM((NG * 6,), jnp.float32),
            pltpu.VMEM((9 * NG,), jnp.float32),
            pltpu.VMEM((RB, DIN), jnp.float32),
            pltpu.VMEM((RB, DIN), jnp.float32),
            pltpu.VMEM((RB, DOUT), jnp.float32),
            pltpu.VMEM((RB, DOUT), jnp.float32),
            pltpu.SemaphoreType.DMA,
            pltpu.SemaphoreType.DMA,
            pltpu.SemaphoreType.DMA,
            pltpu.SemaphoreType.DMA,
        ],
    )
    return run(x, wgts.astype(jnp.float32).reshape(-1))
